# baseline (speedup 1.0000x reference)
import functools

import jax
import jax.numpy as jnp
from jax import lax
from jax.experimental import pallas as pl
from jax.experimental.pallas import tpu as pltpu
from jax.experimental.pallas import tpu_sc as plsc

_NUM_CORES = 2
_NUM_SUBCORES = 16
_NUM_WORKERS = _NUM_CORES * _NUM_SUBCORES


def _sc_gather(ent, rel, nv, h, t, r):
    B = h.shape[0]
    D = ent.shape[1]
    bpw = B // _NUM_WORKERS
    out_t = jax.ShapeDtypeStruct((B, D), jnp.float32)
    mesh = plsc.VectorSubcoreMesh(core_axis_name="c", subcore_axis_name="s")

    @functools.partial(
        pl.kernel,
        mesh=mesh,
        compiler_params=pltpu.CompilerParams(use_tc_tiling_on_sc=False),
        out_type=(out_t, out_t, out_t, out_t),
        scratch_types=[
            pltpu.VMEM((bpw,), jnp.int32),
            pltpu.VMEM((bpw, D), jnp.float32),
            pltpu.SemaphoreType.DMA,
        ],
    )
    def k(ent_hbm, rel_hbm, nv_hbm, h_hbm, t_hbm, r_hbm,
          eh_o, et_o, rr_o, nn_o, idx_v, rows_v, sem):
        wid = lax.axis_index("s") * _NUM_CORES + lax.axis_index("c")
        base = wid * bpw
        sl = pl.ds(base, bpw)

        pltpu.sync_copy(h_hbm.at[sl], idx_v)
        pltpu.async_copy(ent_hbm.at[idx_v], rows_v, sem).wait()
        pltpu.sync_copy(rows_v, eh_o.at[sl])

        pltpu.sync_copy(t_hbm.at[sl], idx_v)
        pltpu.async_copy(ent_hbm.at[idx_v], rows_v, sem).wait()
        pltpu.sync_copy(rows_v, et_o.at[sl])

        pltpu.sync_copy(r_hbm.at[sl], idx_v)
        pltpu.async_copy(rel_hbm.at[idx_v], rows_v, sem).wait()
        pltpu.sync_copy(rows_v, rr_o.at[sl])
        pltpu.async_copy(nv_hbm.at[idx_v], rows_v, sem).wait()
        pltpu.sync_copy(rows_v, nn_o.at[sl])

    return k(ent, rel, nv, h, t, r)


def _tc_math(eh, et, rr, nn):
    B, D = eh.shape
    BT = 2048

    def body(eh_ref, et_ref, rr_ref, nn_ref, o_ref):
        dv = eh_ref[...] - et_ref[...]
        n = nn_ref[...]
        s = jnp.sum(dv * n, axis=1, keepdims=True)
        o_ref[...] = jnp.abs(dv + rr_ref[...] - s * n)

    return pl.pallas_call(
        body,
        grid=(B // BT,),
        in_specs=[pl.BlockSpec((BT, D), lambda i: (i, 0))] * 4,
        out_specs=pl.BlockSpec((BT, D), lambda i: (i, 0)),
        out_shape=jax.ShapeDtypeStruct((B, D), jnp.float32),
    )(eh, et, rr, nn)


def kernel(h, t, r, ent_embeddings, rel_embeddings, normal_vectors):
    h = h.astype(jnp.int32)
    t = t.astype(jnp.int32)
    r = r.astype(jnp.int32)
    eh, et, rr, nn = _sc_gather(
        ent_embeddings, rel_embeddings, normal_vectors, h, t, r)
    return _tc_math(eh, et, rr, nn)
